# Initial kernel scaffold; baseline (speedup 1.0000x reference)
import jax
import jax.numpy as jnp
from jax.experimental import pallas as pl
from jax.experimental.pallas import tpu as pltpu

_N = 5000
_PAD = 5120
_R = 40
_C = 128


def _loss_kernel(pred_ref, true_ref, out_ref):
    px = pred_ref[0]
    py = pred_ref[1]
    pz = pred_ref[2]
    pp = pred_ref[3]
    occ_p = pp >= 0.5
    b2 = px * px + py * py + pz * pz
    lin = (jax.lax.broadcasted_iota(jnp.int32, (_R, _C), 0) * _C
           + jax.lax.broadcasted_iota(jnp.int32, (_R, _C), 1))

    def step(i, carry):
        taken, nt, nm, sdn, smse = carry
        tx = true_ref[i, 0]
        ty = true_ref[i, 1]
        tz = true_ref[i, 2]
        tp = true_ref[i, 3]
        occ_t = tp >= 0.5
        a2 = tx * tx + ty * ty + tz * tz
        dot = tx * px + ty * py + tz * pz
        d2 = (a2 + b2) - 2.0 * dot
        dist = jnp.sqrt(jnp.maximum(d2, 0.0))
        valid = (dist <= 1.0) & occ_p & jnp.logical_not(taken) & occ_t
        r = jnp.where(valid, dist, jnp.inf)
        minval = jnp.min(r)
        matched = minval < jnp.inf
        eq = r == minval
        idx = jnp.min(jnp.where(eq, lin, jnp.int32(2 ** 30)))
        m1 = lin == idx
        taken = jnp.logical_or(taken, m1 & matched)
        zero = jnp.float32(0.0)
        pxi = jnp.sum(jnp.where(m1, px, zero))
        pyi = jnp.sum(jnp.where(m1, py, zero))
        pzi = jnp.sum(jnp.where(m1, pz, zero))
        ppi = jnp.sum(jnp.where(m1, pp, zero))
        dx = tx - pxi
        dy = ty - pyi
        dz = tz - pzi
        dn = jnp.sqrt(dx * dx + dy * dy + dz * dz)
        mf = jnp.where(matched, jnp.float32(1.0), zero)
        nt = nt + jnp.where(occ_t, jnp.float32(1.0), zero)
        nm = nm + mf
        sdn = sdn + mf * dn
        dp = tp - ppi
        smse = smse + mf * dp * dp
        return taken, nt, nm, sdn, smse

    taken0 = jnp.zeros((_R, _C), dtype=jnp.bool_)
    z = jnp.float32(0.0)
    taken, nt, nm, sdn, smse = jax.lax.fori_loop(
        0, _N, step, (taken0, z, z, z, z))
    nu = nt - nm
    denom = jnp.maximum(nm, 1.0)
    extra = jnp.where(nm > 0.0, sdn / denom + smse / denom, 0.0)
    out_ref[0, 0] = 10.0 * nu + nu + extra


def kernel(pred_cloud, true_cloud):
    pt = jnp.transpose(pred_cloud)
    pad = jnp.full((4, _PAD - _N), -1.0, dtype=pred_cloud.dtype)
    pred_pad = jnp.concatenate([pt, pad], axis=1).reshape(4, _R, _C)
    out = pl.pallas_call(
        _loss_kernel,
        out_shape=jax.ShapeDtypeStruct((1, 1), jnp.float32),
    )(pred_pad, true_cloud)
    return out.reshape(())


# speedup vs baseline: 11.1730x; 11.1730x over previous
import jax
import jax.numpy as jnp
from jax.experimental import pallas as pl
from jax.experimental.pallas import tpu as pltpu

_N = 5000
_PAD = 5120
_R = 40
_C = 128


def _loss_kernel(pred_ref, true_ref, out_ref):
    px = pred_ref[0]
    py = pred_ref[1]
    pz = pred_ref[2]
    pp = pred_ref[3]
    occ_p = pp >= 0.5
    b2 = px * px + py * py + pz * pz
    lin = (jax.lax.broadcasted_iota(jnp.int32, (_R, _C), 0) * _C
           + jax.lax.broadcasted_iota(jnp.int32, (_R, _C), 1))

    def step(i, carry):
        taken, nt, nm, sdn, smse = carry
        tx = true_ref[i, 0]
        ty = true_ref[i, 1]
        tz = true_ref[i, 2]
        tp = true_ref[i, 3]
        occ_t = tp >= 0.5
        a2 = tx * tx + ty * ty + tz * tz
        dot = tx * px + ty * py + tz * pz
        d2 = (a2 + b2) - 2.0 * dot
        dist = jnp.sqrt(jnp.maximum(d2, 0.0))
        valid = (dist <= 1.0) & occ_p & (taken == 0.0) & occ_t
        r = jnp.where(valid, dist, jnp.inf)
        minval = jnp.min(r)
        matched = minval < jnp.inf
        eq = r == minval
        idx = jnp.min(jnp.where(eq, lin, jnp.int32(2 ** 30)))
        m1 = lin == idx
        taken = jnp.where(m1 & matched, jnp.float32(1.0), taken)
        zero = jnp.float32(0.0)
        pxi = jnp.sum(jnp.where(m1, px, zero))
        pyi = jnp.sum(jnp.where(m1, py, zero))
        pzi = jnp.sum(jnp.where(m1, pz, zero))
        ppi = jnp.sum(jnp.where(m1, pp, zero))
        dx = tx - pxi
        dy = ty - pyi
        dz = tz - pzi
        dn = jnp.sqrt(dx * dx + dy * dy + dz * dz)
        mf = jnp.where(matched, jnp.float32(1.0), zero)
        nt = nt + jnp.where(occ_t, jnp.float32(1.0), zero)
        nm = nm + mf
        sdn = sdn + mf * dn
        dp = tp - ppi
        smse = smse + mf * dp * dp
        return taken, nt, nm, sdn, smse

    taken0 = jnp.zeros((_R, _C), dtype=jnp.float32)
    z = jnp.float32(0.0)
    taken, nt, nm, sdn, smse = jax.lax.fori_loop(
        0, _N, step, (taken0, z, z, z, z))
    nu = nt - nm
    denom = jnp.maximum(nm, 1.0)
    extra = jnp.where(nm > 0.0, sdn / denom + smse / denom, 0.0)
    out_ref[0, 0] = 10.0 * nu + nu + extra


def kernel(pred_cloud, true_cloud):
    pt = jnp.transpose(pred_cloud)
    pad = jnp.full((4, _PAD - _N), -1.0, dtype=pred_cloud.dtype)
    pred_pad = jnp.concatenate([pt, pad], axis=1).reshape(4, _R, _C)
    out = pl.pallas_call(
        _loss_kernel,
        out_shape=jax.ShapeDtypeStruct((1, 1), jnp.float32),
        out_specs=pl.BlockSpec(memory_space=pltpu.SMEM),
    )(pred_pad, true_cloud)
    return out.reshape(())


# speedup vs baseline: 16.9240x; 1.5147x over previous
import jax
import jax.numpy as jnp
from jax.experimental import pallas as pl
from jax.experimental.pallas import tpu as pltpu

_N = 5000
_PAD = 5120
_R = 40
_C = 128


def _loss_kernel(pred_ref, praw_ref, true_ref, tprob_ref, out_ref):
    px = pred_ref[0]
    py = pred_ref[1]
    pz = pred_ref[2]
    pp = pred_ref[3]
    occ_p = pp >= 0.5
    b2 = px * px + py * py + pz * pz
    b2 = jnp.where(occ_p, b2, jnp.float32(1e9))
    lin = (jax.lax.broadcasted_iota(jnp.int32, (_R, _C), 0) * _C
           + jax.lax.broadcasted_iota(jnp.int32, (_R, _C), 1))
    nt = jnp.sum(jnp.where(tprob_ref[0] >= 0.5, jnp.float32(1.0), jnp.float32(0.0)))

    def occupied_step(i, carry):
        taken, nm, sdn, smse = carry
        tx = true_ref[i, 0]
        ty = true_ref[i, 1]
        tz = true_ref[i, 2]
        tp = true_ref[i, 3]
        a2 = tx * tx + ty * ty + tz * tz
        dot = tx * px + ty * py + tz * pz
        d2 = (a2 + b2) - 2.0 * dot
        dist = jnp.sqrt(jnp.maximum(d2, 0.0))
        avail = (dist <= 1.0) & (taken == 0.0)
        r = jnp.where(avail, dist, jnp.inf)
        minval = jnp.min(r)
        matched = minval < jnp.inf
        eq = r == minval
        idx = jnp.min(jnp.where(eq, lin, jnp.int32(2 ** 30)))
        m1 = lin == idx
        taken = jnp.where(m1 & matched, jnp.float32(1.0), taken)
        pxi = praw_ref[idx, 0]
        pyi = praw_ref[idx, 1]
        pzi = praw_ref[idx, 2]
        ppi = praw_ref[idx, 3]
        dx = tx - pxi
        dy = ty - pyi
        dz = tz - pzi
        dn = jnp.sqrt(dx * dx + dy * dy + dz * dz)
        mf = jnp.where(matched, jnp.float32(1.0), jnp.float32(0.0))
        nm = nm + mf
        sdn = sdn + mf * dn
        dp = tp - ppi
        smse = smse + mf * dp * dp
        return taken, nm, sdn, smse

    def step(i, carry):
        tp = true_ref[i, 3]
        return jax.lax.cond(tp >= 0.5,
                            lambda c: occupied_step(i, c),
                            lambda c: c,
                            carry)

    taken0 = jnp.zeros((_R, _C), dtype=jnp.float32)
    z = jnp.float32(0.0)
    taken, nm, sdn, smse = jax.lax.fori_loop(
        0, _N, step, (taken0, z, z, z))
    nu = nt - nm
    denom = jnp.maximum(nm, 1.0)
    extra = jnp.where(nm > 0.0, sdn / denom + smse / denom, 0.0)
    out_ref[0, 0] = 10.0 * nu + nu + extra


def kernel(pred_cloud, true_cloud):
    pt = jnp.transpose(pred_cloud)
    pad = jnp.full((4, _PAD - _N), -1.0, dtype=pred_cloud.dtype)
    pred_pad = jnp.concatenate([pt, pad], axis=1).reshape(4, _R, _C)
    tprob = jnp.concatenate(
        [true_cloud[:, 3], jnp.full((_PAD - _N,), -1.0, dtype=true_cloud.dtype)]
    ).reshape(1, _R, _C)
    out = pl.pallas_call(
        _loss_kernel,
        out_shape=jax.ShapeDtypeStruct((1, 1), jnp.float32),
        out_specs=pl.BlockSpec(memory_space=pltpu.SMEM),
    )(pred_pad, pred_cloud, true_cloud, tprob)
    return out.reshape(())


# speedup vs baseline: 35.3417x; 2.0883x over previous
import jax
import jax.numpy as jnp
from jax.experimental import pallas as pl
from jax.experimental.pallas import tpu as pltpu

_N = 5000
_PAD = 5120
_R = 40
_C = 128


def _loss_kernel(pred_ref, praw_ref, traw_ref, tprob_ref, out_ref):
    px = pred_ref[0]
    py = pred_ref[1]
    pz = pred_ref[2]
    pp = pred_ref[3]
    occ_p = pp >= 0.5
    b2 = px * px + py * py + pz * pz
    b2 = jnp.where(occ_p, b2, jnp.float32(1e9))
    lin = (jax.lax.broadcasted_iota(jnp.int32, (_R, _C), 0) * _C
           + jax.lax.broadcasted_iota(jnp.int32, (_R, _C), 1))
    nt = jnp.sum(jnp.where(tprob_ref[0] >= 0.5, jnp.float32(1.0), jnp.float32(0.0)))

    def occupied_step(i, carry):
        taken, nm, sdn, smse = carry
        base = i * 4
        tx = traw_ref[base]
        ty = traw_ref[base + 1]
        tz = traw_ref[base + 2]
        tp = traw_ref[base + 3]
        a2 = tx * tx + ty * ty + tz * tz
        dot = tx * px + ty * py + tz * pz
        d2 = (a2 + b2) - 2.0 * dot
        dist = jnp.sqrt(jnp.maximum(d2, 0.0))
        avail = (dist <= 1.0) & (taken == 0.0)
        r = jnp.where(avail, dist, jnp.inf)
        mv = jnp.min(r, axis=0, keepdims=True)
        mb = jnp.broadcast_to(jnp.min(mv, axis=1, keepdims=True), (_R, _C))
        eq = r == mb
        idxv = jnp.where(eq, lin, jnp.int32(2 ** 30))
        iv = jnp.min(idxv, axis=0, keepdims=True)
        ib = jnp.broadcast_to(jnp.min(iv, axis=1, keepdims=True), (_R, _C))
        m1 = (lin == ib) & (mb < jnp.inf)
        taken = jnp.where(m1, jnp.float32(1.0), taken)
        minval = mb[0, 0]
        matched = minval < jnp.inf
        idx = ib[0, 0]
        pbase = idx * 4
        pxi = praw_ref[pbase]
        pyi = praw_ref[pbase + 1]
        pzi = praw_ref[pbase + 2]
        ppi = praw_ref[pbase + 3]
        dx = tx - pxi
        dy = ty - pyi
        dz = tz - pzi
        dn = jnp.sqrt(dx * dx + dy * dy + dz * dz)
        mf = jnp.where(matched, jnp.float32(1.0), jnp.float32(0.0))
        nm = nm + mf
        sdn = sdn + mf * dn
        dp = tp - ppi
        smse = smse + mf * dp * dp
        return taken, nm, sdn, smse

    def step(i, carry):
        tp = traw_ref[i * 4 + 3]
        return jax.lax.cond(tp >= 0.5,
                            lambda c: occupied_step(i, c),
                            lambda c: c,
                            carry)

    taken0 = jnp.zeros((_R, _C), dtype=jnp.float32)
    z = jnp.float32(0.0)
    taken, nm, sdn, smse = jax.lax.fori_loop(
        0, _N, step, (taken0, z, z, z))
    nu = nt - nm
    denom = jnp.maximum(nm, 1.0)
    extra = jnp.where(nm > 0.0, sdn / denom + smse / denom, 0.0)
    out_ref[0, 0] = 10.0 * nu + nu + extra


def kernel(pred_cloud, true_cloud):
    pt = jnp.transpose(pred_cloud)
    pad = jnp.full((4, _PAD - _N), -1.0, dtype=pred_cloud.dtype)
    pred_pad = jnp.concatenate([pt, pad], axis=1).reshape(4, _R, _C)
    tprob = jnp.concatenate(
        [true_cloud[:, 3], jnp.full((_PAD - _N,), -1.0, dtype=true_cloud.dtype)]
    ).reshape(1, _R, _C)
    out = pl.pallas_call(
        _loss_kernel,
        out_shape=jax.ShapeDtypeStruct((1, 1), jnp.float32),
        in_specs=[
            pl.BlockSpec(memory_space=pltpu.VMEM),
            pl.BlockSpec(memory_space=pltpu.SMEM),
            pl.BlockSpec(memory_space=pltpu.SMEM),
            pl.BlockSpec(memory_space=pltpu.VMEM),
        ],
        out_specs=pl.BlockSpec(memory_space=pltpu.SMEM),
    )(pred_pad, pred_cloud.reshape(-1), true_cloud.reshape(-1), tprob)
    return out.reshape(())


# speedup vs baseline: 69.0821x; 1.9547x over previous
import jax
import jax.numpy as jnp
from jax.experimental import pallas as pl
from jax.experimental.pallas import tpu as pltpu

_N = 5000
_PAD = 5120
_R = 40
_C = 128
_CH = 8
_BIGIDX = 3e7


def _argmin_pair(r, linf):
    mv = jnp.min(r, axis=0, keepdims=True)
    mb = jnp.broadcast_to(jnp.min(mv, axis=1, keepdims=True), (_R, _C))
    idxf = jnp.where(r == mb, linf, _BIGIDX)
    iv = jnp.min(idxf, axis=0, keepdims=True)
    ivm = jnp.min(iv, axis=1, keepdims=True)
    return mb[0, 0], ivm[0, 0]


def _loss_kernel(pred_ref, praw_ref, traw_ref, tprob_ref, out_ref):
    px = pred_ref[0]
    py = pred_ref[1]
    pz = pred_ref[2]
    pp = pred_ref[3]
    occ_p = pp >= 0.5
    b2 = px * px + py * py + pz * pz
    b2 = jnp.where(occ_p, b2, jnp.float32(1e9))
    lini = (jax.lax.broadcasted_iota(jnp.int32, (_R, _C), 0) * _C
            + jax.lax.broadcasted_iota(jnp.int32, (_R, _C), 1))
    linf = lini.astype(jnp.float32)
    nt = jnp.sum(jnp.where(tprob_ref[0] >= 0.5, jnp.float32(1.0), jnp.float32(0.0)))

    def chunk(c, carry):
        takenv, nm, sdn, smse = carry
        base = c * (_CH * 4)
        spec = []
        for k in range(_CH):
            off = base + 4 * k
            tx = traw_ref[off]
            ty = traw_ref[off + 1]
            tz = traw_ref[off + 2]
            tp = traw_ref[off + 3]
            occpen = jnp.where(tp >= 0.5, jnp.float32(0.0), jnp.inf)
            a2 = tx * tx + ty * ty + tz * tz + occpen
            dot = tx * px + ty * py + tz * pz
            d2 = (a2 + b2) - 2.0 * dot
            dist = jnp.sqrt(jnp.maximum(d2, 0.0))
            r0 = jnp.where(dist <= 1.0, dist, jnp.inf)
            minval, idxs = _argmin_pair(r0 + takenv, linf)
            spec.append((minval, idxs, r0, tx, ty, tz, tp))

        bs = []
        for k in range(_CH):
            minval, idxs, r0, tx, ty, tz, tp = spec[k]
            coll = jnp.bool_(False)
            for j in range(k):
                coll = jnp.logical_or(coll, idxs == bs[j])

            def redo(tv, r0=r0):
                return _argmin_pair(r0 + tv, linf)

            def keep(tv, mv=minval, ix=idxs):
                return mv, ix

            minval, idxs = jax.lax.cond(coll, redo, keep, takenv)
            matched = minval < jnp.inf
            b_k = jnp.where(matched, idxs, jnp.float32(-1.0))
            bs.append(b_k)
            takenv = jnp.where(linf == b_k, jnp.inf, takenv)
            idxi = jnp.maximum(b_k, jnp.float32(0.0)).astype(jnp.int32)
            pbase = idxi * 4
            pxi = praw_ref[pbase]
            pyi = praw_ref[pbase + 1]
            pzi = praw_ref[pbase + 2]
            ppi = praw_ref[pbase + 3]
            dx = tx - pxi
            dy = ty - pyi
            dz = tz - pzi
            dn = jnp.sqrt(dx * dx + dy * dy + dz * dz)
            mf = jnp.where(matched, jnp.float32(1.0), jnp.float32(0.0))
            nm = nm + mf
            sdn = sdn + mf * dn
            dp = tp - ppi
            smse = smse + mf * dp * dp
        return takenv, nm, sdn, smse

    taken0 = jnp.zeros((_R, _C), dtype=jnp.float32)
    z = jnp.float32(0.0)
    takenv, nm, sdn, smse = jax.lax.fori_loop(
        0, _N // _CH, chunk, (taken0, z, z, z))
    nu = nt - nm
    denom = jnp.maximum(nm, 1.0)
    extra = jnp.where(nm > 0.0, sdn / denom + smse / denom, 0.0)
    out_ref[0, 0] = 10.0 * nu + nu + extra


def kernel(pred_cloud, true_cloud):
    pt = jnp.transpose(pred_cloud)
    pad = jnp.full((4, _PAD - _N), -1.0, dtype=pred_cloud.dtype)
    pred_pad = jnp.concatenate([pt, pad], axis=1).reshape(4, _R, _C)
    tprob = jnp.concatenate(
        [true_cloud[:, 3], jnp.full((_PAD - _N,), -1.0, dtype=true_cloud.dtype)]
    ).reshape(1, _R, _C)
    out = pl.pallas_call(
        _loss_kernel,
        out_shape=jax.ShapeDtypeStruct((1, 1), jnp.float32),
        in_specs=[
            pl.BlockSpec(memory_space=pltpu.VMEM),
            pl.BlockSpec(memory_space=pltpu.SMEM),
            pl.BlockSpec(memory_space=pltpu.SMEM),
            pl.BlockSpec(memory_space=pltpu.VMEM),
        ],
        out_specs=pl.BlockSpec(memory_space=pltpu.SMEM),
    )(pred_pad, pred_cloud.reshape(-1), true_cloud.reshape(-1), tprob)
    return out.reshape(())


# speedup vs baseline: 103.7037x; 1.5012x over previous
import jax
import jax.numpy as jnp
from jax.experimental import pallas as pl
from jax.experimental.pallas import tpu as pltpu

_N = 5000
_PAD = 5120
_R = 40
_C = 128
_CH = 8
_BIGIDX = 3e7


def _loss_kernel(pred_ref, praw_ref, traw_ref, tprob_ref, out_ref):
    px = pred_ref[0]
    py = pred_ref[1]
    pz = pred_ref[2]
    pp = pred_ref[3]
    occ_p = pp >= 0.5
    b2 = px * px + py * py + pz * pz
    b2 = jnp.where(occ_p, b2, jnp.float32(1e9))
    lini = (jax.lax.broadcasted_iota(jnp.int32, (_R, _C), 0) * _C
            + jax.lax.broadcasted_iota(jnp.int32, (_R, _C), 1))
    linf = lini.astype(jnp.float32)
    nt = jnp.sum(jnp.where(tprob_ref[0] >= 0.5, jnp.float32(1.0), jnp.float32(0.0)))

    def argmin_pair(r):
        mv = jnp.min(r, axis=0, keepdims=True)
        mb = jnp.broadcast_to(jnp.min(mv, axis=1, keepdims=True), (_R, _C))
        idxf = jnp.where(r == mb, linf, _BIGIDX)
        iv = jnp.min(idxf, axis=0, keepdims=True)
        ivm = jnp.min(iv, axis=1, keepdims=True)
        return mb[0, 0], ivm[0, 0]

    def dist_row(off):
        tx = traw_ref[off]
        ty = traw_ref[off + 1]
        tz = traw_ref[off + 2]
        tp = traw_ref[off + 3]
        occpen = jnp.where(tp >= 0.5, jnp.float32(0.0), jnp.inf)
        a2 = tx * tx + ty * ty + tz * tz + occpen
        dot = tx * px + ty * py + tz * pz
        d2 = (a2 + b2) - 2.0 * dot
        dist = jnp.sqrt(jnp.maximum(d2, 0.0))
        return jnp.where(dist <= 1.0, dist, jnp.inf)

    def accum(k, base, b_k, matched, nm, sdn, smse):
        off = base + 4 * k
        tx = traw_ref[off]
        ty = traw_ref[off + 1]
        tz = traw_ref[off + 2]
        tp = traw_ref[off + 3]
        idxi = jnp.maximum(b_k, jnp.float32(0.0)).astype(jnp.int32)
        pbase = idxi * 4
        pxi = praw_ref[pbase]
        pyi = praw_ref[pbase + 1]
        pzi = praw_ref[pbase + 2]
        ppi = praw_ref[pbase + 3]
        dx = tx - pxi
        dy = ty - pyi
        dz = tz - pzi
        dn = jnp.sqrt(dx * dx + dy * dy + dz * dz)
        mf = jnp.where(matched, jnp.float32(1.0), jnp.float32(0.0))
        dp = tp - ppi
        return nm + mf, sdn + mf * dn, smse + mf * dp * dp

    def chunk(c, carry):
        base = c * (_CH * 4)
        takenv0, nm0, sdn0, smse0 = carry
        spec = []
        for k in range(_CH):
            spec.append(argmin_pair(dist_row(base + 4 * k) + takenv0))

        mats = [mv < jnp.inf for mv, _ in spec]
        bsp = [jnp.where(mats[k], spec[k][1], jnp.float32(-1.0))
               for k in range(_CH)]
        coll = jnp.bool_(False)
        for k in range(1, _CH):
            ck = jnp.bool_(False)
            for j in range(k):
                ck = jnp.logical_or(ck, spec[k][1] == bsp[j])
            coll = jnp.logical_or(coll, jnp.logical_and(mats[k], ck))

        def fast(op):
            takenv, nm, sdn, smse = op
            masks = [linf == bsp[k] for k in range(_CH)]
            while len(masks) > 1:
                nxt = [masks[i] | masks[i + 1] for i in range(0, len(masks) - 1, 2)]
                if len(masks) % 2:
                    nxt.append(masks[-1])
                masks = nxt
            takenv = jnp.where(masks[0], jnp.inf, takenv)
            for k in range(_CH):
                nm, sdn, smse = accum(k, base, bsp[k], mats[k], nm, sdn, smse)
            return takenv, nm, sdn, smse

        def slow(op):
            takenv, nm, sdn, smse = op
            bs = []
            for k in range(_CH):
                mv, ix = spec[k]
                ck = jnp.bool_(False)
                for j in range(k):
                    ck = jnp.logical_or(ck, ix == bs[j])
                ck = jnp.logical_and(mats[k], ck)

                def redo(tv, kk=k):
                    return argmin_pair(dist_row(base + 4 * kk) + tv)

                def keep(tv, mv=mv, ix=ix):
                    return mv, ix

                mv, ix = jax.lax.cond(ck, redo, keep, takenv)
                matched = mv < jnp.inf
                b_k = jnp.where(matched, ix, jnp.float32(-1.0))
                bs.append(b_k)
                takenv = jnp.where(linf == b_k, jnp.inf, takenv)
                nm, sdn, smse = accum(k, base, b_k, matched, nm, sdn, smse)
            return takenv, nm, sdn, smse

        return jax.lax.cond(coll, slow, fast,
                            (takenv0, nm0, sdn0, smse0))

    taken0 = jnp.zeros((_R, _C), dtype=jnp.float32)
    z = jnp.float32(0.0)
    takenv, nm, sdn, smse = jax.lax.fori_loop(
        0, _N // _CH, chunk, (taken0, z, z, z))
    nu = nt - nm
    denom = jnp.maximum(nm, 1.0)
    extra = jnp.where(nm > 0.0, sdn / denom + smse / denom, 0.0)
    out_ref[0, 0] = 10.0 * nu + nu + extra


def kernel(pred_cloud, true_cloud):
    pt = jnp.transpose(pred_cloud)
    pad = jnp.full((4, _PAD - _N), -1.0, dtype=pred_cloud.dtype)
    pred_pad = jnp.concatenate([pt, pad], axis=1).reshape(4, _R, _C)
    tprob = jnp.concatenate(
        [true_cloud[:, 3], jnp.full((_PAD - _N,), -1.0, dtype=true_cloud.dtype)]
    ).reshape(1, _R, _C)
    out = pl.pallas_call(
        _loss_kernel,
        out_shape=jax.ShapeDtypeStruct((1, 1), jnp.float32),
        in_specs=[
            pl.BlockSpec(memory_space=pltpu.VMEM),
            pl.BlockSpec(memory_space=pltpu.SMEM),
            pl.BlockSpec(memory_space=pltpu.SMEM),
            pl.BlockSpec(memory_space=pltpu.VMEM),
        ],
        out_specs=pl.BlockSpec(memory_space=pltpu.SMEM),
    )(pred_pad, pred_cloud.reshape(-1), true_cloud.reshape(-1), tprob)
    return out.reshape(())
